# baseline (speedup 1.0000x reference)
import functools
import math

import jax
import jax.numpy as jnp
from jax import lax
from jax.experimental import pallas as pl
from jax.experimental.pallas import tpu as pltpu
from jax.experimental.pallas import tpu_sc as plsc

D_MODEL = 64
SCALE = math.sqrt(D_MODEL)
LANES = 16
NUM_CORES = 2
NUM_SUBCORES = 16
NUM_WORKERS = NUM_CORES * NUM_SUBCORES
GROUP = 128
CHUNK_ROWS = 1024
GROUPS_PER_CHUNK = CHUNK_ROWS // GROUP


def kernel(x, table):
    b0, s = x.shape
    batch = b0 * s
    rows_per_worker = batch // NUM_WORKERS
    chunks_per_worker = rows_per_worker // CHUNK_ROWS
    groups_per_worker = rows_per_worker // GROUP

    idx2d = x.reshape(batch // GROUP, GROUP).astype(jnp.int32)

    mesh = plsc.VectorSubcoreMesh(core_axis_name="c", subcore_axis_name="s")

    @functools.partial(
        pl.kernel,
        out_type=jax.ShapeDtypeStruct((batch, D_MODEL), jnp.float32),
        mesh=mesh,
        scratch_types=[
            pltpu.VMEM((GROUPS_PER_CHUNK, GROUP), jnp.int32),
            pltpu.VMEM((CHUNK_ROWS, D_MODEL), jnp.float32),
            pltpu.SemaphoreType.DMA,
        ],
        compiler_params=pltpu.CompilerParams(use_tc_tiling_on_sc=False),
    )
    def emb_kernel(idx_hbm, table_hbm, out_hbm, idx_v, rows_v, sem):
        wid = lax.axis_index("s") * NUM_CORES + lax.axis_index("c")
        base_group = wid * groups_per_worker

        def chunk_body(c, carry):
            g0 = base_group + c * GROUPS_PER_CHUNK
            pltpu.sync_copy(idx_hbm.at[pl.ds(g0, GROUPS_PER_CHUNK)], idx_v)
            copies = []
            for j in range(GROUPS_PER_CHUNK):
                copies.append(
                    pltpu.async_copy(
                        table_hbm.at[idx_v.at[j]],
                        rows_v.at[pl.ds(j * GROUP, GROUP)],
                        sem,
                    )
                )
            for cp in copies:
                cp.wait()

            def row_body(r, rc):
                for k in range(D_MODEL // LANES):
                    sl = pl.ds(k * LANES, LANES)
                    rows_v[r, sl] = rows_v[r, sl] * SCALE
                return rc

            lax.fori_loop(0, CHUNK_ROWS, row_body, 0)

            row0 = g0 * GROUP
            pltpu.sync_copy(rows_v, out_hbm.at[pl.ds(row0, CHUNK_ROWS)])
            return carry

        lax.fori_loop(0, chunks_per_worker, chunk_body, 0)

    out = emb_kernel(idx2d, table)
    return out.reshape(b0, s, D_MODEL)


# speedup vs baseline: 1.1124x; 1.1124x over previous
import functools
import math

import jax
import jax.numpy as jnp
from jax import lax
from jax.experimental import pallas as pl
from jax.experimental.pallas import tpu as pltpu
from jax.experimental.pallas import tpu_sc as plsc

D_MODEL = 64
SCALE = math.sqrt(D_MODEL)
LANES = 16
NUM_CORES = 2
NUM_SUBCORES = 16
NUM_WORKERS = NUM_CORES * NUM_SUBCORES
GROUP = 128
GROUPS_PER_SUB = 4
SUB_ROWS = GROUP * GROUPS_PER_SUB


def kernel(x, table):
    b0, s = x.shape
    batch = b0 * s
    rows_per_worker = batch // NUM_WORKERS
    nsub = rows_per_worker // SUB_ROWS
    nsub_total = batch // SUB_ROWS

    idx3d = x.reshape(nsub_total, GROUPS_PER_SUB, GROUP).astype(jnp.int32)

    mesh = plsc.VectorSubcoreMesh(core_axis_name="c", subcore_axis_name="s")

    @functools.partial(
        pl.kernel,
        out_type=jax.ShapeDtypeStruct((batch, D_MODEL), jnp.float32),
        mesh=mesh,
        scratch_types=[
            pltpu.VMEM((GROUPS_PER_SUB, GROUP), jnp.int32),
            pltpu.VMEM((GROUPS_PER_SUB, GROUP), jnp.int32),
            pltpu.VMEM((SUB_ROWS, D_MODEL), jnp.float32),
            pltpu.VMEM((SUB_ROWS, D_MODEL), jnp.float32),
            pltpu.SemaphoreType.DMA,
            pltpu.SemaphoreType.DMA,
            pltpu.SemaphoreType.DMA,
            pltpu.SemaphoreType.DMA,
            pltpu.SemaphoreType.DMA,
            pltpu.SemaphoreType.DMA,
        ],
        compiler_params=pltpu.CompilerParams(use_tc_tiling_on_sc=False),
    )
    def emb_kernel(idx_hbm, table_hbm, out_hbm,
                   idx_v0, idx_v1, rows_v0, rows_v1,
                   i_sem0, i_sem1, g_sem0, g_sem1, o_sem0, o_sem1):
        idx_v = (idx_v0, idx_v1)
        rows_v = (rows_v0, rows_v1)
        i_sem = (i_sem0, i_sem1)
        g_sem = (g_sem0, g_sem1)
        o_sem = (o_sem0, o_sem1)

        wid = lax.axis_index("s") * NUM_CORES + lax.axis_index("c")
        base_sub = wid * nsub

        def fire_gathers(t, slot):
            for j in range(GROUPS_PER_SUB):
                pltpu.async_copy(
                    table_hbm.at[idx_v[slot].at[j]],
                    rows_v[slot].at[pl.ds(j * GROUP, GROUP)],
                    g_sem[slot],
                )

        def wait_gathers(slot):
            pltpu.make_async_copy(
                table_hbm.at[pl.ds(0, SUB_ROWS)], rows_v[slot], g_sem[slot]
            ).wait()

        def start_idx(t, slot):
            pltpu.async_copy(idx_hbm.at[t], idx_v[slot], i_sem[slot])

        def wait_idx(slot):
            pltpu.make_async_copy(
                idx_hbm.at[0], idx_v[slot], i_sem[slot]
            ).wait()

        def scale_rows(slot):
            buf = rows_v[slot]

            @plsc.parallel_loop(0, SUB_ROWS, unroll=8)
            def _(r):
                for k in range(D_MODEL // LANES):
                    sl = pl.ds(k * LANES, LANES)
                    buf[r, sl] = buf[r, sl] * SCALE

        def fire_out(t, slot):
            pltpu.async_copy(
                rows_v[slot],
                out_hbm.at[pl.ds(t * SUB_ROWS, SUB_ROWS)],
                o_sem[slot],
            )

        def wait_out(slot):
            pltpu.make_async_copy(
                rows_v[slot], out_hbm.at[pl.ds(0, SUB_ROWS)], o_sem[slot]
            ).wait()

        pltpu.sync_copy(idx_hbm.at[base_sub], idx_v[0])
        fire_gathers(base_sub, 0)
        start_idx(base_sub + 1, 1)
        wait_idx(1)
        fire_gathers(base_sub + 1, 1)
        wait_gathers(0)
        start_idx(base_sub + 2, 0)
        scale_rows(0)
        fire_out(base_sub, 0)

        def steady(t, slot):
            other = 1 - slot
            wait_out(other)
            wait_idx(other)
            fire_gathers(t + 1, other)
            wait_gathers(slot)

            @pl.when(t + 2 < base_sub + nsub)
            def _():
                start_idx(t + 2, slot)

            scale_rows(slot)
            fire_out(t, slot)

        @pl.loop(0, (nsub - 2) // 2)
        def _(i):
            t = base_sub + 1 + i * 2
            steady(t, 1)
            steady(t + 1, 0)

        t_last = base_sub + nsub - 1
        wait_out(0)
        wait_gathers(1)
        scale_rows(1)
        fire_out(t_last, 1)
        wait_out(1)

    out = emb_kernel(idx3d, table)
    return out.reshape(b0, s, D_MODEL)
